# baseline (speedup 1.0000x reference)
import functools

import jax
import jax.numpy as jnp
from jax.experimental import pallas as pl

B = 16384
FD = 64
CTX = 128
RD = 256
BB = 1024


def _mlp_body(idx_ref, logit_ref, mol_ref, tt_ref, pt_ref, gt_ref, rt_ref,
              w1e_ref, wlog_ref, w1m_ref, b1_ref, w2_ref, b2_ref, out_ref):
    f32 = jnp.float32
    idx = idx_ref[...]
    iota16 = jax.lax.broadcasted_iota(jnp.int32, (BB, 16), 1)
    iota8 = jax.lax.broadcasted_iota(jnp.int32, (BB, 8), 1)
    oh_t = (idx[:, 0:1] == iota16).astype(f32)
    oh_p = (idx[:, 1:2] == iota8).astype(f32)
    oh_g = (idx[:, 2:3] == iota8).astype(f32)
    oh_r = (idx[:, 3:4] == iota8).astype(f32)

    p_t = jnp.dot(tt_ref[...], w1e_ref[0:64, :], preferred_element_type=f32)
    p_p = jnp.dot(pt_ref[...], w1e_ref[64:128, :], preferred_element_type=f32)
    p_g = jnp.dot(gt_ref[...], w1e_ref[128:192, :], preferred_element_type=f32)
    p_r = jnp.dot(rt_ref[...], w1e_ref[192:256, :], preferred_element_type=f32)

    acc = jnp.dot(mol_ref[...], w1m_ref[...], preferred_element_type=f32)
    acc = acc + jnp.dot(oh_t, p_t, preferred_element_type=f32)
    acc = acc + jnp.dot(oh_p, p_p, preferred_element_type=f32)
    acc = acc + jnp.dot(oh_g, p_g, preferred_element_type=f32)
    acc = acc + jnp.dot(oh_r, p_r, preferred_element_type=f32)
    acc = acc + logit_ref[...] * wlog_ref[...]
    acc = acc + b1_ref[...]
    h = 0.5 * acc * (1.0 + jax.lax.erf(acc * 0.7071067811865476))
    out_ref[...] = jnp.dot(h, w2_ref[...], preferred_element_type=f32) + b2_ref[...]


@jax.jit
def _run(idx_cat, logit2d, mol_repr, type_table, prep_table, geom_table,
         read_table, w1_emb, w_log, w1_mol, b1_2d, w2, b2_2d):
    nb = B // BB
    return pl.pallas_call(
        _mlp_body,
        grid=(nb,),
        in_specs=[
            pl.BlockSpec((BB, 4), lambda i: (i, 0)),
            pl.BlockSpec((BB, 1), lambda i: (i, 0)),
            pl.BlockSpec((BB, RD), lambda i: (i, 0)),
            pl.BlockSpec((16, FD), lambda i: (0, 0)),
            pl.BlockSpec((8, FD), lambda i: (0, 0)),
            pl.BlockSpec((8, FD), lambda i: (0, 0)),
            pl.BlockSpec((8, FD), lambda i: (0, 0)),
            pl.BlockSpec((4 * FD, CTX), lambda i: (0, 0)),
            pl.BlockSpec((1, CTX), lambda i: (0, 0)),
            pl.BlockSpec((RD, CTX), lambda i: (0, 0)),
            pl.BlockSpec((1, CTX), lambda i: (0, 0)),
            pl.BlockSpec((CTX, CTX), lambda i: (0, 0)),
            pl.BlockSpec((1, CTX), lambda i: (0, 0)),
        ],
        out_specs=pl.BlockSpec((BB, CTX), lambda i: (i, 0)),
        out_shape=jax.ShapeDtypeStruct((B, CTX), jnp.float32),
    )(idx_cat, logit2d, mol_repr, type_table, prep_table, geom_table,
      read_table, w1_emb, w_log, w1_mol, b1_2d, w2, b2_2d)


def kernel(assay_type_idx, assay_prep_idx, assay_geometry_idx, assay_readout_idx,
           binding_logit, mol_repr, type_table, prep_table, geom_table, read_table,
           W1, b1, W2, b2):
    idx_cat = jnp.stack(
        [assay_type_idx.astype(jnp.int32), assay_prep_idx.astype(jnp.int32),
         assay_geometry_idx.astype(jnp.int32), assay_readout_idx.astype(jnp.int32)],
        axis=1)
    logit2d = binding_logit.reshape(B, 1)
    w1_emb = W1[0:4 * FD]
    w_log = W1[4 * FD:4 * FD + 1]
    w1_mol = W1[4 * FD + 1:]
    return _run(idx_cat, logit2d, mol_repr, type_table, prep_table, geom_table,
                read_table, w1_emb, w_log, w1_mol, b1.reshape(1, CTX), W2,
                b2.reshape(1, CTX))


# speedup vs baseline: 1.1638x; 1.1638x over previous
import functools

import jax
import jax.numpy as jnp
from jax.experimental import pallas as pl

B = 16384
FD = 64
CTX = 128
RD = 256
BB = 4096


def _mlp_body(idx_ref, logit_ref, mol_ref, tt_ref, pt_ref, gt_ref, rt_ref,
              w1e_ref, wlog_ref, w1m_ref, b1_ref, w2_ref, b2_ref, out_ref):
    f32 = jnp.float32
    idx = idx_ref[...]
    iota16 = jax.lax.broadcasted_iota(jnp.int32, (BB, 16), 1)
    iota8 = jax.lax.broadcasted_iota(jnp.int32, (BB, 8), 1)
    oh_t = (idx[:, 0:1] == iota16).astype(f32)
    oh_p = (idx[:, 1:2] == iota8).astype(f32)
    oh_g = (idx[:, 2:3] == iota8).astype(f32)
    oh_r = (idx[:, 3:4] == iota8).astype(f32)

    p_t = jnp.dot(tt_ref[...], w1e_ref[0:64, :], preferred_element_type=f32)
    p_p = jnp.dot(pt_ref[...], w1e_ref[64:128, :], preferred_element_type=f32)
    p_g = jnp.dot(gt_ref[...], w1e_ref[128:192, :], preferred_element_type=f32)
    p_r = jnp.dot(rt_ref[...], w1e_ref[192:256, :], preferred_element_type=f32)

    acc = jnp.dot(mol_ref[...], w1m_ref[...], preferred_element_type=f32)
    acc = acc + jnp.dot(oh_t, p_t, preferred_element_type=f32)
    acc = acc + jnp.dot(oh_p, p_p, preferred_element_type=f32)
    acc = acc + jnp.dot(oh_g, p_g, preferred_element_type=f32)
    acc = acc + jnp.dot(oh_r, p_r, preferred_element_type=f32)
    acc = acc + logit_ref[...] * wlog_ref[...]
    acc = acc + b1_ref[...]
    h = 0.5 * acc * (1.0 + jax.lax.erf(acc * 0.7071067811865476))
    out_ref[...] = jnp.dot(h, w2_ref[...], preferred_element_type=f32) + b2_ref[...]


@jax.jit
def _run(idx_cat, logit2d, mol_repr, type_table, prep_table, geom_table,
         read_table, w1_emb, w_log, w1_mol, b1_2d, w2, b2_2d):
    nb = B // BB
    return pl.pallas_call(
        _mlp_body,
        grid=(nb,),
        in_specs=[
            pl.BlockSpec((BB, 4), lambda i: (i, 0)),
            pl.BlockSpec((BB, 1), lambda i: (i, 0)),
            pl.BlockSpec((BB, RD), lambda i: (i, 0)),
            pl.BlockSpec((16, FD), lambda i: (0, 0)),
            pl.BlockSpec((8, FD), lambda i: (0, 0)),
            pl.BlockSpec((8, FD), lambda i: (0, 0)),
            pl.BlockSpec((8, FD), lambda i: (0, 0)),
            pl.BlockSpec((4 * FD, CTX), lambda i: (0, 0)),
            pl.BlockSpec((1, CTX), lambda i: (0, 0)),
            pl.BlockSpec((RD, CTX), lambda i: (0, 0)),
            pl.BlockSpec((1, CTX), lambda i: (0, 0)),
            pl.BlockSpec((CTX, CTX), lambda i: (0, 0)),
            pl.BlockSpec((1, CTX), lambda i: (0, 0)),
        ],
        out_specs=pl.BlockSpec((BB, CTX), lambda i: (i, 0)),
        out_shape=jax.ShapeDtypeStruct((B, CTX), jnp.float32),
    )(idx_cat, logit2d, mol_repr, type_table, prep_table, geom_table,
      read_table, w1_emb, w_log, w1_mol, b1_2d, w2, b2_2d)


def kernel(assay_type_idx, assay_prep_idx, assay_geometry_idx, assay_readout_idx,
           binding_logit, mol_repr, type_table, prep_table, geom_table, read_table,
           W1, b1, W2, b2):
    idx_cat = jnp.stack(
        [assay_type_idx.astype(jnp.int32), assay_prep_idx.astype(jnp.int32),
         assay_geometry_idx.astype(jnp.int32), assay_readout_idx.astype(jnp.int32)],
        axis=1)
    logit2d = binding_logit.reshape(B, 1)
    w1_emb = W1[0:4 * FD]
    w_log = W1[4 * FD:4 * FD + 1]
    w1_mol = W1[4 * FD + 1:]
    return _run(idx_cat, logit2d, mol_repr, type_table, prep_table, geom_table,
                read_table, w1_emb, w_log, w1_mol, b1.reshape(1, CTX), W2,
                b2.reshape(1, CTX))
